# baseline (speedup 1.0000x reference)
import functools

import jax
import jax.numpy as jnp
import numpy as np
from jax.experimental import pallas as pl
from jax.experimental.pallas import tpu as pltpu

NBAND = 1
NUM_CODE = 1024
CODE_DIM = 64
EPS = float(np.finfo(np.float32).eps)

TT = 2048
BB = 4


def _vq_body(x_ref, emb_ref, q_ref, idx_ref, perp_ref, acc_ref, *, ng, ntok):
    g = pl.program_id(0)
    first = g == 0
    last = g == ng - 1

    emb = emb_ref[0]
    e2 = jnp.sum(emb * emb, axis=1, keepdims=True)

    @pl.when(first)
    def _():
        acc_ref[...] = jnp.zeros_like(acc_ref)

    for j in range(BB):
        x = x_ref[j, 0]
        x2 = jnp.sum(x * x, axis=0, keepdims=True)
        d = jax.lax.dot_general(emb, x, (((1,), (0,)), ((), ())),
                                preferred_element_type=jnp.float32)
        dist = (x2 * 0.5 + e2 * 0.5) - d

        minv = jnp.min(dist, axis=0, keepdims=True)
        oh0 = (dist == minv).astype(jnp.float32)
        colsum = jnp.sum(oh0, axis=0, keepdims=True)
        tie = jnp.max(colsum) > 1.0

        @pl.when(jnp.logical_not(tie))
        def _(j=j, oh0=oh0):
            kio_col = jax.lax.broadcasted_iota(
                jnp.int32, (NUM_CODE, 1), 0).astype(jnp.float32)
            idx_row = jax.lax.dot_general(
                kio_col, oh0, (((0,), (0,)), ((), ())),
                preferred_element_type=jnp.float32)
            idx_ref[j, 0] = idx_row[0].astype(jnp.int32)
            q = jax.lax.dot_general(emb, oh0, (((0,), (0,)), ((), ())),
                                    preferred_element_type=jnp.float32)
            q_ref[j, 0] = q
            acc_ref[...] += jnp.sum(oh0, axis=1, keepdims=True)

        @pl.when(tie)
        def _(j=j, dist=dist):
            idx_i = jnp.argmin(dist, axis=0)
            idx_ref[j, 0] = idx_i
            kio_i = jax.lax.broadcasted_iota(jnp.int32, dist.shape, 0)
            oh = (kio_i == idx_i[None, :]).astype(jnp.float32)
            q = jax.lax.dot_general(emb, oh, (((0,), (0,)), ((), ())),
                                    preferred_element_type=jnp.float32)
            q_ref[j, 0] = q
            acc_ref[...] += jnp.sum(oh, axis=1, keepdims=True)

    @pl.when(last)
    def _():
        p = acc_ref[...] * (1.0 / ntok)
        ent = jnp.sum(p * jnp.log(p + EPS))
        perp_ref[...] = jnp.full((1, 1), jnp.exp(-ent), dtype=jnp.float32)


@jax.jit
def kernel(input, embedding):
    B, C, N, T = input.shape
    ng = B // BB
    ntok = B * T

    body = functools.partial(_vq_body, ng=ng, ntok=ntok)
    q, idx_raw, perp = pl.pallas_call(
        body,
        grid=(ng,),
        in_specs=[
            pl.BlockSpec((BB, 1, N, TT), lambda g: (g, 0, 0, 0)),
            pl.BlockSpec((NBAND, NUM_CODE, CODE_DIM), lambda g: (0, 0, 0)),
        ],
        out_specs=[
            pl.BlockSpec((BB, 1, N, TT), lambda g: (g, 0, 0, 0)),
            pl.BlockSpec((BB, 1, TT), lambda g: (g, 0, 0)),
            pl.BlockSpec((1, 1), lambda g: (0, 0)),
        ],
        out_shape=[
            jax.ShapeDtypeStruct((B, C, N, T), jnp.float32),
            jax.ShapeDtypeStruct((B, 1, T), jnp.int32),
            jax.ShapeDtypeStruct((1, 1), jnp.float32),
        ],
        scratch_shapes=[pltpu.VMEM((NUM_CODE, 1), jnp.float32)],
        compiler_params=pltpu.CompilerParams(
            dimension_semantics=("arbitrary",),
        ),
    )(input, embedding)

    return q, idx_raw.reshape(B, T, 1), perp.reshape(())


# speedup vs baseline: 1.4235x; 1.4235x over previous
import functools

import jax
import jax.numpy as jnp
import numpy as np
from jax.experimental import pallas as pl
from jax.experimental.pallas import tpu as pltpu

NBAND = 1
NUM_CODE = 1024
CODE_DIM = 64
EPS = float(np.finfo(np.float32).eps)

TT = 2048
BB = 4


def _vq_body(x_ref, emb_ref, q_ref, idx_ref, perp_ref, acc_ref, *, ng, ntok):
    g = pl.program_id(0)
    first = g == 0
    last = g == ng - 1

    emb = emb_ref[0]
    e2 = jnp.sum(emb * emb, axis=1, keepdims=True)

    cnt = acc_ref[...]
    cnt = jnp.where(first, jnp.zeros_like(cnt), cnt)

    for j in range(BB):
        x = x_ref[j, 0]
        x2 = jnp.sum(x * x, axis=0, keepdims=True)
        d = jax.lax.dot_general(emb, x, (((1,), (0,)), ((), ())),
                                preferred_element_type=jnp.float32)
        dist = (x2 * 0.5 + e2 * 0.5) - d

        idx_i = jnp.argmin(dist, axis=0)
        idx_ref[j, 0] = idx_i
        kio_i = jax.lax.broadcasted_iota(jnp.int32, dist.shape, 0)
        oh = (kio_i == idx_i[None, :]).astype(jnp.float32)
        q = jax.lax.dot_general(emb, oh, (((0,), (0,)), ((), ())),
                                preferred_element_type=jnp.float32)
        q_ref[j, 0] = q

        cnt = cnt + jnp.sum(oh, axis=1, keepdims=True)

    acc_ref[...] = cnt

    @pl.when(last)
    def _():
        p = cnt * (1.0 / ntok)
        ent = jnp.sum(p * jnp.log(p + EPS))
        perp_ref[...] = jnp.full((1, 1), jnp.exp(-ent), dtype=jnp.float32)


@jax.jit
def kernel(input, embedding):
    B, C, N, T = input.shape
    ng = B // BB
    ntok = B * T

    body = functools.partial(_vq_body, ng=ng, ntok=ntok)
    q, idx_raw, perp = pl.pallas_call(
        body,
        grid=(ng,),
        in_specs=[
            pl.BlockSpec((BB, 1, N, TT), lambda g: (g, 0, 0, 0)),
            pl.BlockSpec((NBAND, NUM_CODE, CODE_DIM), lambda g: (0, 0, 0)),
        ],
        out_specs=[
            pl.BlockSpec((BB, 1, N, TT), lambda g: (g, 0, 0, 0)),
            pl.BlockSpec((BB, 1, TT), lambda g: (g, 0, 0)),
            pl.BlockSpec((1, 1), lambda g: (0, 0)),
        ],
        out_shape=[
            jax.ShapeDtypeStruct((B, C, N, T), jnp.float32),
            jax.ShapeDtypeStruct((B, 1, T), jnp.int32),
            jax.ShapeDtypeStruct((1, 1), jnp.float32),
        ],
        scratch_shapes=[pltpu.VMEM((NUM_CODE, 1), jnp.float32)],
        compiler_params=pltpu.CompilerParams(
            dimension_semantics=("arbitrary",),
        ),
    )(input, embedding)

    return q, idx_raw.reshape(B, T, 1), perp.reshape(())
